# baseline (speedup 1.0000x reference)
import jax
import jax.numpy as jnp
from jax.experimental import pallas as pl

_H = 64
_G2 = 64
_N = 64
_S = 64
_A = 1024
_CH = 8
_NEIGH = 2.0
_GRID = 8


def _scene_kernel(h_ref, ep_ref, rp_ref, wenc_ref, wdec_ref, wembed_ref,
                  wembatt_ref, wfullt_ref, wout_ref, wmlp_ref, batt_ref,
                  bembed_ref, bout_ref, bmlp_ref, x_ref):
    ch = h_ref[...]
    ep = ep_ref[...]
    rp = rp_ref[...]
    x = ep[:, 0:1]
    y = ep[:, 1:2]
    tlx = x - _NEIGH / 2
    tly = y + _NEIGH / 2
    brx = x + _NEIGH / 2
    bry = y - _NEIGH / 2
    xr = x.reshape(1, _N)
    yr = y.reshape(1, _N)
    cellx = jnp.floor((xr - tlx) / _NEIGH * _GRID)
    celly = jnp.floor((tly - yr) / _NEIGH * _GRID)
    ok = (xr < brx) & (xr > tlx) & (yr < tly) & (yr > bry)
    ii = jax.lax.broadcasted_iota(jnp.int32, (_N, _N), 0)
    jj = jax.lax.broadcasted_iota(jnp.int32, (_N, _N), 1)
    ok = ok & (ii != jj)
    cell = (cellx + float(_GRID) * celly).astype(jnp.int32)
    gi = jax.lax.broadcasted_iota(jnp.int32, (_N, _G2, _N), 1)
    m = jnp.where((cell[:, None, :] == gi) & ok[:, None, :], 1.0, 0.0)
    enc = jnp.dot(m.reshape(_N * _G2, _N), ch,
                  preferred_element_type=jnp.float32)

    emb = jnp.dot(jnp.concatenate([ep, rp], axis=1), wembed_ref[...],
                  preferred_element_type=jnp.float32) + bembed_ref[...]
    c = (jnp.dot(ch, wdec_ref[...], preferred_element_type=jnp.float32)
         + jnp.dot(emb, wembatt_ref[...], preferred_element_type=jnp.float32)
         + batt_ref[...])

    wenc = wenc_ref[...]
    wf = wfullt_ref[...].reshape(1, 1, _A)
    att_rows = []
    for t in range(_N // _CH):
        encc = enc[t * _CH * _G2:(t + 1) * _CH * _G2]
        a1 = jnp.dot(encc, wenc, preferred_element_type=jnp.float32)
        a1 = a1.reshape(_CH, _G2, _A) + c[t * _CH:(t + 1) * _CH][:, None, :]
        att_rows.append(jnp.sum(jnp.maximum(a1, 0.0) * wf, axis=2))
    att = jnp.concatenate(att_rows, axis=0)

    att = att - jnp.max(att, axis=1, keepdims=True)
    e = jnp.exp(att)
    alpha = e / jnp.sum(e, axis=1, keepdims=True)
    enc3 = enc.reshape(_N, _G2, _H)
    awe = jnp.sum(enc3 * alpha[:, :, None], axis=1)

    ph = jnp.dot(jnp.concatenate([awe, ch], axis=1), wout_ref[...],
                 preferred_element_type=jnp.float32) + bout_ref[...]
    x_ref[...] = jnp.dot(ph, wmlp_ref[...],
                         preferred_element_type=jnp.float32) + bmlp_ref[...]


def _bn_kernel(x_ref, g_ref, b_ref, o_ref):
    x = x_ref[...]
    m = jnp.mean(x, axis=0, keepdims=True)
    v = jnp.mean((x - m) ** 2, axis=0, keepdims=True)
    y = (x - m) / jnp.sqrt(v + 1e-5) * g_ref[...] + b_ref[...]
    o_ref[...] = jnp.maximum(y, 0.0)


def kernel(h_states, seq_start_end, end_pos, rel_pos, params):
    del seq_start_end
    h_flat = h_states.reshape(-1, _H)
    p = params
    b_att = (p['b_enc'] + p['b_dec'] + p['b_embatt']).reshape(1, _A)
    wfull_t = p['W_full'].reshape(1, _A)

    rep = lambda s: (0, 0)
    x_pre = pl.pallas_call(
        _scene_kernel,
        grid=(_S,),
        in_specs=[
            pl.BlockSpec((_N, _H), lambda s: (s, 0)),
            pl.BlockSpec((_N, 2), lambda s: (s, 0)),
            pl.BlockSpec((_N, 2), lambda s: (s, 0)),
            pl.BlockSpec((_H, _A), rep),
            pl.BlockSpec((_H, _A), rep),
            pl.BlockSpec((4, 4), rep),
            pl.BlockSpec((4, _A), rep),
            pl.BlockSpec((1, _A), rep),
            pl.BlockSpec((2 * _H, _A), rep),
            pl.BlockSpec((_A, _A), rep),
            pl.BlockSpec((1, _A), rep),
            pl.BlockSpec((1, 4), rep),
            pl.BlockSpec((1, _A), rep),
            pl.BlockSpec((1, _A), rep),
        ],
        out_specs=pl.BlockSpec((_N, _A), lambda s: (s, 0)),
        out_shape=jax.ShapeDtypeStruct((_S * _N, _A), jnp.float32),
    )(h_flat, end_pos, rel_pos, p['W_enc'], p['W_dec'], p['W_embed'],
      p['W_embatt'], wfull_t, p['W_out'], p['W_mlp'], b_att,
      p['b_embed'].reshape(1, 4), p['b_out'].reshape(1, _A),
      p['b_mlp'].reshape(1, _A))

    _CB = 256
    out = pl.pallas_call(
        _bn_kernel,
        grid=(_A // _CB,),
        in_specs=[
            pl.BlockSpec((_S * _N, _CB), lambda c: (0, c)),
            pl.BlockSpec((1, _CB), lambda c: (0, c)),
            pl.BlockSpec((1, _CB), lambda c: (0, c)),
        ],
        out_specs=pl.BlockSpec((_S * _N, _CB), lambda c: (0, c)),
        out_shape=jax.ShapeDtypeStruct((_S * _N, _A), jnp.float32),
    )(x_pre, p['bn_gamma'].reshape(1, _A), p['bn_beta'].reshape(1, _A))
    return out


# speedup vs baseline: 9.6254x; 9.6254x over previous
import jax
import jax.numpy as jnp
from jax.experimental import pallas as pl

_H = 64
_G2 = 64
_N = 64
_S = 64
_A = 1024
_GC = 8
_NEIGH = 2.0
_GRID = 8


def _scene_kernel(h_ref, ep_ref, ept_ref, rp_ref, wenc_ref, wdec_ref,
                  wembed_ref, wembatt_ref, wfullt_ref, wout_ref, wmlp_ref,
                  batt_ref, bembed_ref, bout_ref, bmlp_ref, x_ref):
    ch = h_ref[...]
    ep = ep_ref[...]
    rp = rp_ref[...]
    x = ep[:, 0:1]
    y = ep[:, 1:2]
    xr = ept_ref[0, 0:1, :]
    yr = ept_ref[0, 1:2, :]
    tlx = x - _NEIGH / 2
    tly = y + _NEIGH / 2
    brx = x + _NEIGH / 2
    bry = y - _NEIGH / 2
    cellx = jnp.floor((xr - tlx) / _NEIGH * _GRID)
    celly = jnp.floor((tly - yr) / _NEIGH * _GRID)
    ok = (xr < brx) & (xr > tlx) & (yr < tly) & (yr > bry)
    ii = jax.lax.broadcasted_iota(jnp.int32, (_N, _N), 0)
    jj = jax.lax.broadcasted_iota(jnp.int32, (_N, _N), 1)
    ok = ok & (ii != jj)
    cell = (cellx + float(_GRID) * celly).astype(jnp.int32)
    cell3 = jnp.broadcast_to(cell[None], (_G2, _N, _N))
    ok3 = jnp.broadcast_to(ok[None], (_G2, _N, _N))
    g3 = jax.lax.broadcasted_iota(jnp.int32, (_G2, _N, _N), 0)
    m = jnp.where((cell3 == g3) & ok3, 1.0, 0.0)
    enc = jnp.dot(m.reshape(_G2 * _N, _N), ch,
                  preferred_element_type=jnp.float32)

    emb = jnp.dot(jnp.concatenate([ep, rp], axis=1), wembed_ref[...],
                  preferred_element_type=jnp.float32) + bembed_ref[...]
    c = (jnp.dot(ch, wdec_ref[...], preferred_element_type=jnp.float32)
         + jnp.dot(emb, wembatt_ref[...], preferred_element_type=jnp.float32)
         + batt_ref[...])

    wenc = wenc_ref[...]
    wf = wfullt_ref[...].reshape(1, 1, _A)
    att_rows = []
    for t in range(_G2 // _GC):
        encc = enc[t * _GC * _N:(t + 1) * _GC * _N]
        a1 = jnp.dot(encc, wenc, preferred_element_type=jnp.float32)
        a1 = a1.reshape(_GC, _N, _A) + c[None, :, :]
        att_rows.append(jnp.sum(jnp.maximum(a1, 0.0) * wf, axis=2))
    att_t = jnp.concatenate(att_rows, axis=0)

    att_t = att_t - jnp.max(att_t, axis=0, keepdims=True)
    e = jnp.exp(att_t)
    alpha_t = e / jnp.sum(e, axis=0, keepdims=True)
    enc3 = enc.reshape(_G2, _N, _H)
    awe = jnp.sum(enc3 * alpha_t[:, :, None], axis=0)

    ph = jnp.dot(jnp.concatenate([awe, ch], axis=1), wout_ref[...],
                 preferred_element_type=jnp.float32) + bout_ref[...]
    x_ref[...] = jnp.dot(ph, wmlp_ref[...],
                         preferred_element_type=jnp.float32) + bmlp_ref[...]


def _bn_kernel(x_ref, g_ref, b_ref, o_ref):
    x = x_ref[...]
    m = jnp.mean(x, axis=0, keepdims=True)
    v = jnp.mean((x - m) ** 2, axis=0, keepdims=True)
    y = (x - m) / jnp.sqrt(v + 1e-5) * g_ref[...] + b_ref[...]
    o_ref[...] = jnp.maximum(y, 0.0)


def kernel(h_states, seq_start_end, end_pos, rel_pos, params):
    del seq_start_end
    h_flat = h_states.reshape(-1, _H)
    p = params
    b_att = (p['b_enc'] + p['b_dec'] + p['b_embatt']).reshape(1, _A)
    wfull_t = p['W_full'].reshape(1, _A)
    end_pos_t = end_pos.reshape(_S, _N, 2).transpose(0, 2, 1)

    rep = lambda s: (0, 0)
    x_pre = pl.pallas_call(
        _scene_kernel,
        grid=(_S,),
        in_specs=[
            pl.BlockSpec((_N, _H), lambda s: (s, 0)),
            pl.BlockSpec((_N, 2), lambda s: (s, 0)),
            pl.BlockSpec((1, 2, _N), lambda s: (s, 0, 0)),
            pl.BlockSpec((_N, 2), lambda s: (s, 0)),
            pl.BlockSpec((_H, _A), rep),
            pl.BlockSpec((_H, _A), rep),
            pl.BlockSpec((4, 4), rep),
            pl.BlockSpec((4, _A), rep),
            pl.BlockSpec((1, _A), rep),
            pl.BlockSpec((2 * _H, _A), rep),
            pl.BlockSpec((_A, _A), rep),
            pl.BlockSpec((1, _A), rep),
            pl.BlockSpec((1, 4), rep),
            pl.BlockSpec((1, _A), rep),
            pl.BlockSpec((1, _A), rep),
        ],
        out_specs=pl.BlockSpec((_N, _A), lambda s: (s, 0)),
        out_shape=jax.ShapeDtypeStruct((_S * _N, _A), jnp.float32),
    )(h_flat, end_pos, end_pos_t, rel_pos, p['W_enc'], p['W_dec'],
      p['W_embed'], p['W_embatt'], wfull_t, p['W_out'], p['W_mlp'], b_att,
      p['b_embed'].reshape(1, 4), p['b_out'].reshape(1, _A),
      p['b_mlp'].reshape(1, _A))

    _CB = 256
    out = pl.pallas_call(
        _bn_kernel,
        grid=(_A // _CB,),
        in_specs=[
            pl.BlockSpec((_S * _N, _CB), lambda c: (0, c)),
            pl.BlockSpec((1, _CB), lambda c: (0, c)),
            pl.BlockSpec((1, _CB), lambda c: (0, c)),
        ],
        out_specs=pl.BlockSpec((_S * _N, _CB), lambda c: (0, c)),
        out_shape=jax.ShapeDtypeStruct((_S * _N, _A), jnp.float32),
    )(x_pre, p['bn_gamma'].reshape(1, _A), p['bn_beta'].reshape(1, _A))
    return out
